# Initial kernel scaffold; baseline (speedup 1.0000x reference)
import jax
import jax.numpy as jnp
from jax.experimental import pallas as pl


def kernel(x, edge_index, batch, W1, b1, W2, b2, fc1_W, fc1_b, fc2_W, fc2_b):
    raise NotImplementedError("write your pallas kernel here")


# speedup vs baseline: 15.4482x; 15.4482x over previous
import functools

import jax
import jax.numpy as jnp
from jax import lax
from jax.experimental import pallas as pl
from jax.experimental.pallas import tpu as pltpu
from jax.experimental.pallas import tpu_sc as plsc

_N_CORES = 2
_N_SUB = 16
_N_GRAPHS = 64
_CHUNK = 80
_DEG_W = 16


def _make_sc_degree(n_pad, n_edges):
    ntiles = _N_CORES * _N_SUB
    e_per_tile = n_edges // ntiles
    n_chunks = e_per_tile // _CHUNK
    rps = n_pad // _N_SUB

    mesh = plsc.VectorSubcoreMesh(core_axis_name="c", subcore_axis_name="s")

    @functools.partial(
        pl.kernel,
        out_type=jax.ShapeDtypeStruct((_N_CORES, n_pad, _DEG_W), jnp.float32),
        mesh=mesh,
        compiler_params=pltpu.CompilerParams(use_tc_tiling_on_sc=False),
        scratch_types=[
            pltpu.VMEM((_CHUNK,), jnp.int32),
            pltpu.VMEM((_CHUNK, _DEG_W), jnp.float32),
            pltpu.VMEM_SHARED((n_pad, _DEG_W), jnp.float32),
        ],
    )
    def deg_kernel(dst_hbm, ones_hbm, zero_hbm, out_hbm, didx, ones_v, acc):
        c = lax.axis_index("c")
        s = lax.axis_index("s")
        tid = c * _N_SUB + s
        pltpu.sync_copy(ones_hbm, ones_v)
        pltpu.sync_copy(zero_hbm.at[pl.ds(s * rps, rps)],
                        acc.at[pl.ds(s * rps, rps)])
        plsc.subcore_barrier()
        base = tid * e_per_tile

        def body(i, carry):
            off = base + i * _CHUNK
            pltpu.sync_copy(dst_hbm.at[pl.ds(off, _CHUNK)], didx)
            pltpu.sync_copy(ones_v, acc.at[didx], add=True)
            return carry

        lax.fori_loop(0, n_chunks, body, 0)
        plsc.subcore_barrier()
        pltpu.sync_copy(acc.at[pl.ds(s * rps, rps)],
                        out_hbm.at[c, pl.ds(s * rps, rps)])

    return deg_kernel


def _make_sc_scatter(n_pad, n_edges, d):
    ntiles = _N_CORES * _N_SUB
    e_per_tile = n_edges // ntiles
    n_chunks = e_per_tile // _CHUNK
    rps = n_pad // _N_SUB

    mesh = plsc.VectorSubcoreMesh(core_axis_name="c", subcore_axis_name="s")

    @functools.partial(
        pl.kernel,
        out_type=jax.ShapeDtypeStruct((_N_CORES, n_pad, d), jnp.float32),
        mesh=mesh,
        compiler_params=pltpu.CompilerParams(use_tc_tiling_on_sc=False),
        scratch_types=[
            pltpu.VMEM((_CHUNK,), jnp.int32),
            pltpu.VMEM((_CHUNK,), jnp.int32),
            pltpu.VMEM((_CHUNK, d), jnp.float32),
            pltpu.VMEM_SHARED((n_pad, d), jnp.float32),
            pltpu.SemaphoreType.DMA,
        ],
    )
    def scatter_kernel(src_hbm, dst_hbm, feat_hbm, zero_hbm, out_hbm,
                       sidx, didx, rows, acc, sem):
        c = lax.axis_index("c")
        s = lax.axis_index("s")
        tid = c * _N_SUB + s
        pltpu.sync_copy(zero_hbm.at[pl.ds(s * rps, rps)],
                        acc.at[pl.ds(s * rps, rps)])
        plsc.subcore_barrier()
        base = tid * e_per_tile

        def body(i, carry):
            off = base + i * _CHUNK
            pltpu.sync_copy(src_hbm.at[pl.ds(off, _CHUNK)], sidx)
            pltpu.sync_copy(dst_hbm.at[pl.ds(off, _CHUNK)], didx)
            pltpu.async_copy(feat_hbm.at[sidx], rows, sem).wait()
            pltpu.sync_copy(rows, acc.at[didx], add=True)
            return carry

        lax.fori_loop(0, n_chunks, body, 0)
        plsc.subcore_barrier()
        pltpu.sync_copy(acc.at[pl.ds(s * rps, rps)],
                        out_hbm.at[c, pl.ds(s * rps, rps)])

    return scatter_kernel


def _tc1_body(degp_ref, x_ref, w1_ref, dinv_ref, hs_ref):
    deg = degp_ref[0][:, 0:1] + degp_ref[1][:, 0:1] + 1.0
    dinv = lax.rsqrt(deg)
    h = jnp.dot(x_ref[...], w1_ref[...], preferred_element_type=jnp.float32)
    dinv_ref[...] = dinv
    hs_ref[...] = h * dinv


def _tc2_body(p_ref, hs1_ref, dinv_ref, b1_ref, w2_ref, hs2_ref):
    ssum = p_ref[0] + p_ref[1] + hs1_ref[...]
    dinv = dinv_ref[...]
    h1 = jnp.maximum(ssum * dinv + b1_ref[...], 0.0)
    hs2_ref[...] = jnp.dot(h1, w2_ref[...],
                           preferred_element_type=jnp.float32) * dinv


def _tc3_body(p_ref, hs2_ref, dinv_ref, b2_ref, batch_ref, fc1w_ref,
              fc1b_ref, fc2w_ref, fc2b_ref, out_ref, sums, counts, *,
              n_blocks, blk):
    i = pl.program_id(0)

    @pl.when(i == 0)
    def _init():
        sums[...] = jnp.zeros_like(sums)
        counts[...] = jnp.zeros_like(counts)

    ssum = p_ref[0] + p_ref[1] + hs2_ref[...]
    h2 = jnp.maximum(ssum * dinv_ref[...] + b2_ref[...], 0.0)
    b = batch_ref[0]
    oh = (lax.broadcasted_iota(jnp.int32, (_N_GRAPHS, blk), 0) == b
          ).astype(jnp.float32)
    sums[...] += jnp.dot(oh, h2, preferred_element_type=jnp.float32)
    counts[...] += jnp.sum(oh, axis=1, keepdims=True)

    @pl.when(i == n_blocks - 1)
    def _finish():
        pooled = sums[...] / jnp.maximum(counts[...], 1.0)
        g1 = jnp.maximum(
            jnp.dot(pooled, fc1w_ref[...],
                    preferred_element_type=jnp.float32) + fc1b_ref[...], 0.0)
        z = jnp.dot(g1, fc2w_ref[...],
                    preferred_element_type=jnp.float32) + fc2b_ref[...]
        out_ref[...] = jax.nn.sigmoid(z)


def kernel(x, edge_index, batch, W1, b1, W2, b2, fc1_W, fc1_b, fc2_W, fc2_b):
    n, d_in = x.shape
    n_edges = edge_index.shape[1]
    d1 = W1.shape[1]
    d2 = W2.shape[1]
    blk = 2000
    n_blocks = n // blk

    src = edge_index[0].astype(jnp.int32)
    dst = edge_index[1].astype(jnp.int32)
    batch3d = batch.astype(jnp.int32).reshape(n // blk, 1, blk)

    n_pad = _N_SUB * ((n + 8 * _N_SUB - 1) // (8 * _N_SUB)) * 8
    ones16 = jnp.ones((_CHUNK, _DEG_W), jnp.float32)
    zeros16 = jnp.zeros((n_pad, _DEG_W), jnp.float32)
    zeros1 = jnp.zeros((n_pad, d1), jnp.float32)
    zeros2 = jnp.zeros((n_pad, d2), jnp.float32)

    degp = _make_sc_degree(n_pad, n_edges)(dst, ones16, zeros16)

    dinv, hs1 = pl.pallas_call(
        _tc1_body,
        grid=(n_blocks,),
        in_specs=[
            pl.BlockSpec((_N_CORES, blk, _DEG_W), lambda i: (0, i, 0)),
            pl.BlockSpec((blk, d_in), lambda i: (i, 0)),
            pl.BlockSpec((d_in, d1), lambda i: (0, 0)),
        ],
        out_specs=[
            pl.BlockSpec((blk, 1), lambda i: (i, 0)),
            pl.BlockSpec((blk, d1), lambda i: (i, 0)),
        ],
        out_shape=[
            jax.ShapeDtypeStruct((n, 1), jnp.float32),
            jax.ShapeDtypeStruct((n, d1), jnp.float32),
        ],
    )(degp, x, W1)

    p1 = _make_sc_scatter(n_pad, n_edges, d1)(src, dst, hs1, zeros1)

    hs2 = pl.pallas_call(
        _tc2_body,
        grid=(n_blocks,),
        in_specs=[
            pl.BlockSpec((_N_CORES, blk, d1), lambda i: (0, i, 0)),
            pl.BlockSpec((blk, d1), lambda i: (i, 0)),
            pl.BlockSpec((blk, 1), lambda i: (i, 0)),
            pl.BlockSpec((1, d1), lambda i: (0, 0)),
            pl.BlockSpec((d1, d2), lambda i: (0, 0)),
        ],
        out_specs=pl.BlockSpec((blk, d2), lambda i: (i, 0)),
        out_shape=jax.ShapeDtypeStruct((n, d2), jnp.float32),
    )(p1, hs1, dinv, b1.reshape(1, d1), W2)

    p2 = _make_sc_scatter(n_pad, n_edges, d2)(src, dst, hs2, zeros2)

    d3 = fc1_W.shape[1]
    out = pl.pallas_call(
        functools.partial(_tc3_body, n_blocks=n_blocks, blk=blk),
        grid=(n_blocks,),
        in_specs=[
            pl.BlockSpec((_N_CORES, blk, d2), lambda i: (0, i, 0)),
            pl.BlockSpec((blk, d2), lambda i: (i, 0)),
            pl.BlockSpec((blk, 1), lambda i: (i, 0)),
            pl.BlockSpec((1, d2), lambda i: (0, 0)),
            pl.BlockSpec((1, 1, blk), lambda i: (i, 0, 0)),
            pl.BlockSpec((d2, d3), lambda i: (0, 0)),
            pl.BlockSpec((1, d3), lambda i: (0, 0)),
            pl.BlockSpec((d3, 1), lambda i: (0, 0)),
            pl.BlockSpec((1, 1), lambda i: (0, 0)),
        ],
        out_specs=pl.BlockSpec((_N_GRAPHS, 1), lambda i: (0, 0)),
        out_shape=jax.ShapeDtypeStruct((_N_GRAPHS, 1), jnp.float32),
        scratch_shapes=[
            pltpu.VMEM((_N_GRAPHS, d2), jnp.float32),
            pltpu.VMEM((_N_GRAPHS, 1), jnp.float32),
        ],
    )(p2, hs2, dinv, b2.reshape(1, d2), batch3d, fc1_W,
      fc1_b.reshape(1, d3), fc2_W, fc2_b.reshape(1, 1))

    return out


# speedup vs baseline: 37.3063x; 2.4149x over previous
import functools

import jax
import jax.numpy as jnp
from jax import lax
from jax.experimental import pallas as pl
from jax.experimental.pallas import tpu as pltpu
from jax.experimental.pallas import tpu_sc as plsc

_N_CORES = 2
_N_SUB = 16
_N_GRAPHS = 64
_CHUNK = 80
_DEG_W = 16


def _make_sc_degree(n_pad, n_edges):
    ntiles = _N_CORES * _N_SUB
    e_per_tile = n_edges // ntiles
    n_chunks = e_per_tile // _CHUNK
    rps = n_pad // _N_SUB

    mesh = plsc.VectorSubcoreMesh(core_axis_name="c", subcore_axis_name="s")

    @functools.partial(
        pl.kernel,
        out_type=jax.ShapeDtypeStruct((_N_CORES, n_pad, _DEG_W), jnp.float32),
        mesh=mesh,
        compiler_params=pltpu.CompilerParams(use_tc_tiling_on_sc=False),
        scratch_types=[
            pltpu.VMEM((n_chunks, _CHUNK), jnp.int32),
            pltpu.VMEM((_CHUNK, _DEG_W), jnp.float32),
            pltpu.VMEM_SHARED((n_pad, _DEG_W), jnp.float32),
        ],
    )
    def deg_kernel(dst_hbm, ones_hbm, zero_hbm, out_hbm, didx, ones_v, acc):
        c = lax.axis_index("c")
        s = lax.axis_index("s")
        tid = c * _N_SUB + s
        pltpu.sync_copy(dst_hbm.at[tid], didx)
        pltpu.sync_copy(ones_hbm, ones_v)
        pltpu.sync_copy(zero_hbm.at[pl.ds(s * rps, rps)],
                        acc.at[pl.ds(s * rps, rps)])
        plsc.subcore_barrier()

        def body(i, carry):
            pltpu.sync_copy(ones_v, acc.at[didx.at[i]], add=True)
            return carry

        lax.fori_loop(0, n_chunks, body, 0)
        plsc.subcore_barrier()
        pltpu.sync_copy(acc.at[pl.ds(s * rps, rps)],
                        out_hbm.at[c, pl.ds(s * rps, rps)])

    return deg_kernel


def _make_sc_scatter(n_pad, n_edges, d):
    ntiles = _N_CORES * _N_SUB
    e_per_tile = n_edges // ntiles
    n_chunks = e_per_tile // _CHUNK
    rps = n_pad // _N_SUB

    mesh = plsc.VectorSubcoreMesh(core_axis_name="c", subcore_axis_name="s")

    @functools.partial(
        pl.kernel,
        out_type=jax.ShapeDtypeStruct((_N_CORES, n_pad, d), jnp.float32),
        mesh=mesh,
        compiler_params=pltpu.CompilerParams(use_tc_tiling_on_sc=False),
        scratch_types=[
            pltpu.VMEM((n_chunks, _CHUNK), jnp.int32),
            pltpu.VMEM((n_chunks, _CHUNK), jnp.int32),
            pltpu.VMEM((2, _CHUNK, d), jnp.float32),
            pltpu.VMEM_SHARED((n_pad, d), jnp.float32),
            pltpu.SemaphoreType.DMA((2,)),
        ],
    )
    def scatter_kernel(src_hbm, dst_hbm, feat_hbm, zero_hbm, out_hbm,
                       sidx, didx, rows, acc, sem):
        c = lax.axis_index("c")
        s = lax.axis_index("s")
        tid = c * _N_SUB + s
        pltpu.sync_copy(src_hbm.at[tid], sidx)
        pltpu.sync_copy(dst_hbm.at[tid], didx)
        pltpu.sync_copy(zero_hbm.at[pl.ds(s * rps, rps)],
                        acc.at[pl.ds(s * rps, rps)])
        plsc.subcore_barrier()
        pltpu.async_copy(feat_hbm.at[sidx.at[0]], rows.at[0], sem.at[0])

        def body(i, carry):
            b = lax.rem(i, 2)
            nb = lax.rem(i + 1, 2)

            @pl.when(i + 1 < n_chunks)
            def _fire_next():
                pltpu.async_copy(feat_hbm.at[sidx.at[i + 1]], rows.at[nb],
                                 sem.at[nb])

            pltpu.make_async_copy(feat_hbm.at[sidx.at[i]], rows.at[b],
                                  sem.at[b]).wait()
            pltpu.sync_copy(rows.at[b], acc.at[didx.at[i]], add=True)
            return carry

        lax.fori_loop(0, n_chunks, body, 0)
        plsc.subcore_barrier()
        pltpu.sync_copy(acc.at[pl.ds(s * rps, rps)],
                        out_hbm.at[c, pl.ds(s * rps, rps)])

    return scatter_kernel


def _tc1_body(degp_ref, x_ref, w1_ref, dinv_ref, hs_ref):
    deg = degp_ref[0][:, 0:1] + degp_ref[1][:, 0:1] + 1.0
    dinv = lax.rsqrt(deg)
    h = jnp.dot(x_ref[...], w1_ref[...], preferred_element_type=jnp.float32)
    dinv_ref[...] = dinv
    hs_ref[...] = h * dinv


def _tc2_body(p_ref, hs1_ref, dinv_ref, b1_ref, w2_ref, hs2_ref):
    ssum = p_ref[0] + p_ref[1] + hs1_ref[...]
    dinv = dinv_ref[...]
    h1 = jnp.maximum(ssum * dinv + b1_ref[...], 0.0)
    hs2_ref[...] = jnp.dot(h1, w2_ref[...],
                           preferred_element_type=jnp.float32) * dinv


def _tc3_body(p_ref, hs2_ref, dinv_ref, b2_ref, batch_ref, fc1w_ref,
              fc1b_ref, fc2w_ref, fc2b_ref, out_ref, sums, counts, *,
              n_blocks, blk):
    i = pl.program_id(0)

    @pl.when(i == 0)
    def _init():
        sums[...] = jnp.zeros_like(sums)
        counts[...] = jnp.zeros_like(counts)

    ssum = p_ref[0] + p_ref[1] + hs2_ref[...]
    h2 = jnp.maximum(ssum * dinv_ref[...] + b2_ref[...], 0.0)
    b = batch_ref[0]
    oh = (lax.broadcasted_iota(jnp.int32, (_N_GRAPHS, blk), 0) == b
          ).astype(jnp.float32)
    sums[...] += jnp.dot(oh, h2, preferred_element_type=jnp.float32)
    counts[...] += jnp.sum(oh, axis=1, keepdims=True)

    @pl.when(i == n_blocks - 1)
    def _finish():
        pooled = sums[...] / jnp.maximum(counts[...], 1.0)
        g1 = jnp.maximum(
            jnp.dot(pooled, fc1w_ref[...],
                    preferred_element_type=jnp.float32) + fc1b_ref[...], 0.0)
        z = jnp.dot(g1, fc2w_ref[...],
                    preferred_element_type=jnp.float32) + fc2b_ref[...]
        out_ref[...] = jax.nn.sigmoid(z)


def kernel(x, edge_index, batch, W1, b1, W2, b2, fc1_W, fc1_b, fc2_W, fc2_b):
    n, d_in = x.shape
    n_edges = edge_index.shape[1]
    d1 = W1.shape[1]
    d2 = W2.shape[1]
    blk = 2000
    n_blocks = n // blk

    ntiles = _N_CORES * _N_SUB
    n_chunks = n_edges // (ntiles * _CHUNK)
    src = edge_index[0].astype(jnp.int32).reshape(ntiles, n_chunks, _CHUNK)
    dst = edge_index[1].astype(jnp.int32).reshape(ntiles, n_chunks, _CHUNK)
    batch3d = batch.astype(jnp.int32).reshape(n // blk, 1, blk)

    n_pad = _N_SUB * ((n + 8 * _N_SUB - 1) // (8 * _N_SUB)) * 8
    ones16 = jnp.ones((_CHUNK, _DEG_W), jnp.float32)
    zeros16 = jnp.zeros((n_pad, _DEG_W), jnp.float32)
    zeros1 = jnp.zeros((n_pad, d1), jnp.float32)
    zeros2 = jnp.zeros((n_pad, d2), jnp.float32)

    degp = _make_sc_degree(n_pad, n_edges)(dst, ones16, zeros16)

    dinv, hs1 = pl.pallas_call(
        _tc1_body,
        grid=(n_blocks,),
        in_specs=[
            pl.BlockSpec((_N_CORES, blk, _DEG_W), lambda i: (0, i, 0)),
            pl.BlockSpec((blk, d_in), lambda i: (i, 0)),
            pl.BlockSpec((d_in, d1), lambda i: (0, 0)),
        ],
        out_specs=[
            pl.BlockSpec((blk, 1), lambda i: (i, 0)),
            pl.BlockSpec((blk, d1), lambda i: (i, 0)),
        ],
        out_shape=[
            jax.ShapeDtypeStruct((n, 1), jnp.float32),
            jax.ShapeDtypeStruct((n, d1), jnp.float32),
        ],
    )(degp, x, W1)

    p1 = _make_sc_scatter(n_pad, n_edges, d1)(src, dst, hs1, zeros1)

    hs2 = pl.pallas_call(
        _tc2_body,
        grid=(n_blocks,),
        in_specs=[
            pl.BlockSpec((_N_CORES, blk, d1), lambda i: (0, i, 0)),
            pl.BlockSpec((blk, d1), lambda i: (i, 0)),
            pl.BlockSpec((blk, 1), lambda i: (i, 0)),
            pl.BlockSpec((1, d1), lambda i: (0, 0)),
            pl.BlockSpec((d1, d2), lambda i: (0, 0)),
        ],
        out_specs=pl.BlockSpec((blk, d2), lambda i: (i, 0)),
        out_shape=jax.ShapeDtypeStruct((n, d2), jnp.float32),
    )(p1, hs1, dinv, b1.reshape(1, d1), W2)

    p2 = _make_sc_scatter(n_pad, n_edges, d2)(src, dst, hs2, zeros2)

    d3 = fc1_W.shape[1]
    out = pl.pallas_call(
        functools.partial(_tc3_body, n_blocks=n_blocks, blk=blk),
        grid=(n_blocks,),
        in_specs=[
            pl.BlockSpec((_N_CORES, blk, d2), lambda i: (0, i, 0)),
            pl.BlockSpec((blk, d2), lambda i: (i, 0)),
            pl.BlockSpec((blk, 1), lambda i: (i, 0)),
            pl.BlockSpec((1, d2), lambda i: (0, 0)),
            pl.BlockSpec((1, 1, blk), lambda i: (i, 0, 0)),
            pl.BlockSpec((d2, d3), lambda i: (0, 0)),
            pl.BlockSpec((1, d3), lambda i: (0, 0)),
            pl.BlockSpec((d3, 1), lambda i: (0, 0)),
            pl.BlockSpec((1, 1), lambda i: (0, 0)),
        ],
        out_specs=pl.BlockSpec((_N_GRAPHS, 1), lambda i: (0, 0)),
        out_shape=jax.ShapeDtypeStruct((_N_GRAPHS, 1), jnp.float32),
        scratch_shapes=[
            pltpu.VMEM((_N_GRAPHS, d2), jnp.float32),
            pltpu.VMEM((_N_GRAPHS, 1), jnp.float32),
        ],
    )(p2, hs2, dinv, b2.reshape(1, d2), batch3d, fc1_W,
      fc1_b.reshape(1, d3), fc2_W, fc2_b.reshape(1, 1))

    return out


# speedup vs baseline: 48.3929x; 1.2972x over previous
import functools

import jax
import jax.numpy as jnp
from jax import lax
from jax.experimental import pallas as pl
from jax.experimental.pallas import tpu as pltpu
from jax.experimental.pallas import tpu_sc as plsc

_N_CORES = 2
_N_SUB = 16
_N_GRAPHS = 64
_CHUNK = 400
_DEG_W = 16


def _make_sc_degree(n_pad, n_edges):
    ntiles = _N_CORES * _N_SUB
    e_per_tile = n_edges // ntiles
    n_chunks = e_per_tile // _CHUNK
    rps = n_pad // _N_SUB

    mesh = plsc.VectorSubcoreMesh(core_axis_name="c", subcore_axis_name="s")

    @functools.partial(
        pl.kernel,
        out_type=jax.ShapeDtypeStruct((_N_CORES, n_pad, _DEG_W), jnp.float32),
        mesh=mesh,
        compiler_params=pltpu.CompilerParams(use_tc_tiling_on_sc=False),
        scratch_types=[
            pltpu.VMEM((n_chunks, _CHUNK), jnp.int32),
            pltpu.VMEM((_CHUNK, _DEG_W), jnp.float32),
            pltpu.VMEM_SHARED((n_pad, _DEG_W), jnp.float32),
        ],
    )
    def deg_kernel(dst_hbm, ones_hbm, zero_hbm, out_hbm, didx, ones_v, acc):
        c = lax.axis_index("c")
        s = lax.axis_index("s")
        tid = c * _N_SUB + s
        pltpu.sync_copy(dst_hbm.at[tid], didx)
        pltpu.sync_copy(ones_hbm, ones_v)
        pltpu.sync_copy(zero_hbm.at[pl.ds(s * rps, rps)],
                        acc.at[pl.ds(s * rps, rps)])
        plsc.subcore_barrier()

        def body(i, carry):
            pltpu.sync_copy(ones_v, acc.at[didx.at[i]], add=True)
            return carry

        lax.fori_loop(0, n_chunks, body, 0)
        plsc.subcore_barrier()
        pltpu.sync_copy(acc.at[pl.ds(s * rps, rps)],
                        out_hbm.at[c, pl.ds(s * rps, rps)])

    return deg_kernel


def _make_sc_scatter(n_pad, n_edges, d):
    ntiles = _N_CORES * _N_SUB
    e_per_tile = n_edges // ntiles
    n_chunks = e_per_tile // _CHUNK
    rps = n_pad // _N_SUB

    mesh = plsc.VectorSubcoreMesh(core_axis_name="c", subcore_axis_name="s")

    @functools.partial(
        pl.kernel,
        out_type=jax.ShapeDtypeStruct((_N_CORES, n_pad, d), jnp.float32),
        mesh=mesh,
        compiler_params=pltpu.CompilerParams(use_tc_tiling_on_sc=False),
        scratch_types=[
            pltpu.VMEM((n_chunks, _CHUNK), jnp.int32),
            pltpu.VMEM((n_chunks, _CHUNK), jnp.int32),
            pltpu.VMEM((2, _CHUNK, d), jnp.float32),
            pltpu.VMEM_SHARED((n_pad, d), jnp.float32),
            pltpu.SemaphoreType.DMA((2,)),
        ],
    )
    def scatter_kernel(src_hbm, dst_hbm, feat_hbm, zero_hbm, out_hbm,
                       sidx, didx, rows, acc, sem):
        c = lax.axis_index("c")
        s = lax.axis_index("s")
        tid = c * _N_SUB + s
        pltpu.sync_copy(src_hbm.at[tid], sidx)
        pltpu.sync_copy(dst_hbm.at[tid], didx)
        pltpu.sync_copy(zero_hbm.at[pl.ds(s * rps, rps)],
                        acc.at[pl.ds(s * rps, rps)])
        plsc.subcore_barrier()
        pltpu.async_copy(feat_hbm.at[sidx.at[0]], rows.at[0], sem.at[0])

        def body(i, carry):
            b = lax.rem(i, 2)
            nb = lax.rem(i + 1, 2)

            @pl.when(i + 1 < n_chunks)
            def _fire_next():
                pltpu.async_copy(feat_hbm.at[sidx.at[i + 1]], rows.at[nb],
                                 sem.at[nb])

            pltpu.make_async_copy(feat_hbm.at[sidx.at[i]], rows.at[b],
                                  sem.at[b]).wait()
            pltpu.sync_copy(rows.at[b], acc.at[didx.at[i]], add=True)
            return carry

        lax.fori_loop(0, n_chunks, body, 0)
        plsc.subcore_barrier()
        pltpu.sync_copy(acc.at[pl.ds(s * rps, rps)],
                        out_hbm.at[c, pl.ds(s * rps, rps)])

    return scatter_kernel


def _tc1_body(degp_ref, x_ref, w1_ref, dinv_ref, hs_ref):
    deg = degp_ref[0][:, 0:1] + degp_ref[1][:, 0:1] + 1.0
    dinv = lax.rsqrt(deg)
    h = jnp.dot(x_ref[...], w1_ref[...], preferred_element_type=jnp.float32)
    dinv_ref[...] = dinv
    hs_ref[...] = h * dinv


def _tc2_body(p_ref, hs1_ref, dinv_ref, b1_ref, w2_ref, hs2_ref):
    ssum = p_ref[0] + p_ref[1] + hs1_ref[...]
    dinv = dinv_ref[...]
    h1 = jnp.maximum(ssum * dinv + b1_ref[...], 0.0)
    hs2_ref[...] = jnp.dot(h1, w2_ref[...],
                           preferred_element_type=jnp.float32) * dinv


def _tc3_body(p_ref, hs2_ref, dinv_ref, b2_ref, batch_ref, fc1w_ref,
              fc1b_ref, fc2w_ref, fc2b_ref, out_ref, sums, counts, *,
              n_blocks, blk):
    i = pl.program_id(0)

    @pl.when(i == 0)
    def _init():
        sums[...] = jnp.zeros_like(sums)
        counts[...] = jnp.zeros_like(counts)

    ssum = p_ref[0] + p_ref[1] + hs2_ref[...]
    h2 = jnp.maximum(ssum * dinv_ref[...] + b2_ref[...], 0.0)
    b = batch_ref[0]
    oh = (lax.broadcasted_iota(jnp.int32, (_N_GRAPHS, blk), 0) == b
          ).astype(jnp.float32)
    sums[...] += jnp.dot(oh, h2, preferred_element_type=jnp.float32)
    counts[...] += jnp.sum(oh, axis=1, keepdims=True)

    @pl.when(i == n_blocks - 1)
    def _finish():
        pooled = sums[...] / jnp.maximum(counts[...], 1.0)
        g1 = jnp.maximum(
            jnp.dot(pooled, fc1w_ref[...],
                    preferred_element_type=jnp.float32) + fc1b_ref[...], 0.0)
        z = jnp.dot(g1, fc2w_ref[...],
                    preferred_element_type=jnp.float32) + fc2b_ref[...]
        out_ref[...] = jax.nn.sigmoid(z)


def kernel(x, edge_index, batch, W1, b1, W2, b2, fc1_W, fc1_b, fc2_W, fc2_b):
    n, d_in = x.shape
    n_edges = edge_index.shape[1]
    d1 = W1.shape[1]
    d2 = W2.shape[1]
    blk = 2000
    n_blocks = n // blk

    ntiles = _N_CORES * _N_SUB
    n_chunks = n_edges // (ntiles * _CHUNK)
    src = edge_index[0].astype(jnp.int32).reshape(ntiles, n_chunks, _CHUNK)
    dst = edge_index[1].astype(jnp.int32).reshape(ntiles, n_chunks, _CHUNK)
    batch3d = batch.astype(jnp.int32).reshape(n // blk, 1, blk)

    n_pad = _N_SUB * ((n + 8 * _N_SUB - 1) // (8 * _N_SUB)) * 8
    ones16 = jnp.ones((_CHUNK, _DEG_W), jnp.float32)
    zeros16 = jnp.zeros((n_pad, _DEG_W), jnp.float32)
    zeros1 = jnp.zeros((n_pad, d1), jnp.float32)
    zeros2 = jnp.zeros((n_pad, d2), jnp.float32)

    degp = _make_sc_degree(n_pad, n_edges)(dst, ones16, zeros16)

    dinv, hs1 = pl.pallas_call(
        _tc1_body,
        grid=(n_blocks,),
        in_specs=[
            pl.BlockSpec((_N_CORES, blk, _DEG_W), lambda i: (0, i, 0)),
            pl.BlockSpec((blk, d_in), lambda i: (i, 0)),
            pl.BlockSpec((d_in, d1), lambda i: (0, 0)),
        ],
        out_specs=[
            pl.BlockSpec((blk, 1), lambda i: (i, 0)),
            pl.BlockSpec((blk, d1), lambda i: (i, 0)),
        ],
        out_shape=[
            jax.ShapeDtypeStruct((n, 1), jnp.float32),
            jax.ShapeDtypeStruct((n, d1), jnp.float32),
        ],
    )(degp, x, W1)

    p1 = _make_sc_scatter(n_pad, n_edges, d1)(src, dst, hs1, zeros1)

    hs2 = pl.pallas_call(
        _tc2_body,
        grid=(n_blocks,),
        in_specs=[
            pl.BlockSpec((_N_CORES, blk, d1), lambda i: (0, i, 0)),
            pl.BlockSpec((blk, d1), lambda i: (i, 0)),
            pl.BlockSpec((blk, 1), lambda i: (i, 0)),
            pl.BlockSpec((1, d1), lambda i: (0, 0)),
            pl.BlockSpec((d1, d2), lambda i: (0, 0)),
        ],
        out_specs=pl.BlockSpec((blk, d2), lambda i: (i, 0)),
        out_shape=jax.ShapeDtypeStruct((n, d2), jnp.float32),
    )(p1, hs1, dinv, b1.reshape(1, d1), W2)

    p2 = _make_sc_scatter(n_pad, n_edges, d2)(src, dst, hs2, zeros2)

    d3 = fc1_W.shape[1]
    out = pl.pallas_call(
        functools.partial(_tc3_body, n_blocks=n_blocks, blk=blk),
        grid=(n_blocks,),
        in_specs=[
            pl.BlockSpec((_N_CORES, blk, d2), lambda i: (0, i, 0)),
            pl.BlockSpec((blk, d2), lambda i: (i, 0)),
            pl.BlockSpec((blk, 1), lambda i: (i, 0)),
            pl.BlockSpec((1, d2), lambda i: (0, 0)),
            pl.BlockSpec((1, 1, blk), lambda i: (i, 0, 0)),
            pl.BlockSpec((d2, d3), lambda i: (0, 0)),
            pl.BlockSpec((1, d3), lambda i: (0, 0)),
            pl.BlockSpec((d3, 1), lambda i: (0, 0)),
            pl.BlockSpec((1, 1), lambda i: (0, 0)),
        ],
        out_specs=pl.BlockSpec((_N_GRAPHS, 1), lambda i: (0, 0)),
        out_shape=jax.ShapeDtypeStruct((_N_GRAPHS, 1), jnp.float32),
        scratch_shapes=[
            pltpu.VMEM((_N_GRAPHS, d2), jnp.float32),
            pltpu.VMEM((_N_GRAPHS, 1), jnp.float32),
        ],
    )(p2, hs2, dinv, b2.reshape(1, d2), batch3d, fc1_W,
      fc1_b.reshape(1, d3), fc2_W, fc2_b.reshape(1, 1))

    return out
